# baseline (speedup 1.0000x reference)
import functools

import jax
import jax.numpy as jnp
from jax import lax
from jax.experimental import pallas as pl
from jax.experimental.pallas import tpu as pltpu
from jax.experimental.pallas import tpu_sc as plsc

_L = 16

_DNUMS = lax.GatherDimensionNumbers(
    offset_dims=(), collapsed_slice_dims=(0,), start_index_map=(0,))


def _lanesum(x):
    for k in range(4):
        perm = lax.iota(jnp.int32, _L) ^ (1 << k)
        x = x + lax.gather(x, perm[:, None], _DNUMS, slice_sizes=(1,),
                           mode=lax.GatherScatterMode.PROMISE_IN_BOUNDS)
    return x


def _normalize_body(w_ref, n_ref):
    w = w_ref[...]
    denom = jnp.maximum(jnp.sqrt(jnp.sum(w * w, axis=1, keepdims=True)), 1e-12)
    n_ref[...] = w / denom


def _normalize_rows(W_w):
    return pl.pallas_call(
        _normalize_body,
        out_shape=jax.ShapeDtypeStruct(W_w.shape, W_w.dtype),
    )(W_w)


def _make_sc_kernel(batch, dim, chunk):
    info = plsc.get_sparse_core_info()
    nc, ns = info.num_cores, info.num_subcores
    nw = nc * ns
    per_tile = batch // nw
    n_chunks = per_tile // chunk
    nvec = dim // _L
    mesh = plsc.VectorSubcoreMesh(core_axis_name="c", subcore_axis_name="s")

    @functools.partial(
        pl.kernel,
        out_type=jax.ShapeDtypeStruct((batch,), jnp.float32),
        mesh=mesh,
        compiler_params=pltpu.CompilerParams(use_tc_tiling_on_sc=False),
        scratch_types=[
            pltpu.VMEM((chunk,), jnp.int32),
            pltpu.VMEM((chunk,), jnp.int32),
            pltpu.VMEM((chunk,), jnp.int32),
            pltpu.VMEM((chunk, dim), jnp.float32),
            pltpu.VMEM((chunk, dim), jnp.float32),
            pltpu.VMEM((chunk, dim), jnp.float32),
            pltpu.VMEM((chunk, dim), jnp.float32),
            pltpu.VMEM((chunk,), jnp.float32),
            pltpu.SemaphoreType.DMA,
        ],
    )
    def sc_kernel(h_hbm, t_hbm, r_hbm, e_hbm, rel_hbm, wn_hbm, out_hbm,
                  hidx, tidx, ridx, hrows, trows, rrows, nrows, outbuf, sem):
        wid = lax.axis_index("s") * nc + lax.axis_index("c")
        base = wid * per_tile
        lane = lax.iota(jnp.int32, _L)
        for c in range(n_chunks):
            off = base + c * chunk
            pltpu.sync_copy(h_hbm.at[pl.ds(off, chunk)], hidx)
            pltpu.sync_copy(t_hbm.at[pl.ds(off, chunk)], tidx)
            pltpu.sync_copy(r_hbm.at[pl.ds(off, chunk)], ridx)
            cps = [
                pltpu.async_copy(e_hbm.at[hidx], hrows, sem),
                pltpu.async_copy(e_hbm.at[tidx], trows, sem),
                pltpu.async_copy(rel_hbm.at[ridx], rrows, sem),
                pltpu.async_copy(wn_hbm.at[ridx], nrows, sem),
            ]
            for cp in cps:
                cp.wait()

            def group(g, _):
                vec = None
                for j in range(_L):
                    i = g * _L + j
                    d = [hrows[i, pl.ds(k * _L, _L)] - trows[i, pl.ds(k * _L, _L)]
                         for k in range(nvec)]
                    n = [nrows[i, pl.ds(k * _L, _L)] for k in range(nvec)]
                    p = d[0] * n[0]
                    for k in range(1, nvec):
                        p = p + d[k] * n[k]
                    s = _lanesum(p)
                    acc = None
                    for k in range(nvec):
                        v = d[k] - s * n[k] + rrows[i, pl.ds(k * _L, _L)]
                        a = jnp.abs(v)
                        acc = a if acc is None else acc + a
                    tot = _lanesum(acc)
                    vec = tot if vec is None else jnp.where(lane == j, tot, vec)
                outbuf[pl.ds(g * _L, _L)] = vec
                return 0

            lax.fori_loop(0, chunk // _L, group, 0)
            pltpu.sync_copy(outbuf, out_hbm.at[pl.ds(off, chunk)])

    return sc_kernel


def kernel(h, r, t, E_w, R_w, W_w):
    batch = h.shape[0]
    dim = E_w.shape[1]
    W_n = _normalize_rows(W_w)
    sc = _make_sc_kernel(batch, dim, chunk=128)
    return sc(h, t, r, E_w, R_w, W_n)


# speedup vs baseline: 1.6449x; 1.6449x over previous
import functools

import jax
import jax.numpy as jnp
from jax import lax
from jax.experimental import pallas as pl
from jax.experimental.pallas import tpu as pltpu
from jax.experimental.pallas import tpu_sc as plsc

_L = 16

_DNUMS = lax.GatherDimensionNumbers(
    offset_dims=(), collapsed_slice_dims=(0,), start_index_map=(0,))


def _lanesum(x):
    for k in range(4):
        perm = lax.iota(jnp.int32, _L) ^ (1 << k)
        x = x + lax.gather(x, perm[:, None], _DNUMS, slice_sizes=(1,),
                           mode=lax.GatherScatterMode.PROMISE_IN_BOUNDS)
    return x


def _fuse_body(r_ref, w_ref, out_ref):
    w = w_ref[...]
    denom = jnp.maximum(jnp.sqrt(jnp.sum(w * w, axis=1, keepdims=True)), 1e-12)
    out_ref[...] = jnp.concatenate([r_ref[...], w / denom], axis=1)


def _fuse_relations(R_w, W_w):
    n_rel, dim = R_w.shape
    return pl.pallas_call(
        _fuse_body,
        out_shape=jax.ShapeDtypeStruct((n_rel, 2 * dim), R_w.dtype),
    )(R_w, W_w)


def _make_sc_kernel(batch, dim, chunk):
    info = plsc.get_sparse_core_info()
    nc, ns = info.num_cores, info.num_subcores
    nw = nc * ns
    per_tile = batch // nw
    n_chunks = per_tile // chunk
    nvec = dim // _L
    mesh = plsc.VectorSubcoreMesh(core_axis_name="c", subcore_axis_name="s")

    @functools.partial(
        pl.kernel,
        out_type=jax.ShapeDtypeStruct((batch,), jnp.float32),
        mesh=mesh,
        scratch_types=[
            pltpu.VMEM_SHARED((ns, chunk), jnp.int32),
            pltpu.VMEM_SHARED((ns, chunk), jnp.int32),
            pltpu.SMEM((chunk,), jnp.int32),
            pltpu.SMEM((chunk,), jnp.int32),
            pltpu.VMEM((chunk,), jnp.int32),
            pltpu.VMEM((chunk, dim), jnp.float32),
            pltpu.VMEM((chunk, dim), jnp.float32),
            pltpu.VMEM((chunk, 2 * dim), jnp.float32),
            pltpu.VMEM((chunk,), jnp.float32),
            pltpu.SemaphoreType.DMA,
            pltpu.SemaphoreType.DMA,
        ],
    )
    def sc_kernel(h_hbm, t_hbm, r_hbm, e_hbm, rw_hbm, out_hbm,
                  sh_h, sh_t, hs, ts, ridx, hrows, trows, rwrows, outbuf,
                  sem, rsem):
        sid = lax.axis_index("s")
        wid = sid * nc + lax.axis_index("c")
        base = wid * per_tile
        lane = lax.iota(jnp.int32, _L)
        for c in range(n_chunks):
            off = base + c * chunk
            pltpu.sync_copy(r_hbm.at[pl.ds(off, chunk)], ridx)
            rw_cp = pltpu.async_copy(rw_hbm.at[ridx], rwrows, sem)
            pltpu.sync_copy(h_hbm.at[pl.ds(off, chunk)], sh_h.at[sid])
            pltpu.sync_copy(t_hbm.at[pl.ds(off, chunk)], sh_t.at[sid])
            pltpu.sync_copy(sh_h.at[sid], hs)
            pltpu.sync_copy(sh_t.at[sid], ts)

            def fetch(i, _):
                pltpu.async_copy(e_hbm.at[hs[i]], hrows.at[i], rsem)
                pltpu.async_copy(e_hbm.at[ts[i]], trows.at[i], rsem)
                return 0

            lax.fori_loop(0, chunk, fetch, 0)
            pltpu.make_async_copy(e_hbm.at[pl.ds(0, chunk)], hrows, rsem).wait()
            pltpu.make_async_copy(e_hbm.at[pl.ds(0, chunk)], trows, rsem).wait()
            rw_cp.wait()

            def group(g, _):
                vec = None
                for j in range(_L):
                    i = g * _L + j
                    d = [hrows[i, pl.ds(k * _L, _L)] - trows[i, pl.ds(k * _L, _L)]
                         for k in range(nvec)]
                    n = [rwrows[i, pl.ds(dim + k * _L, _L)] for k in range(nvec)]
                    p = d[0] * n[0]
                    for k in range(1, nvec):
                        p = p + d[k] * n[k]
                    s = _lanesum(p)
                    acc = None
                    for k in range(nvec):
                        v = d[k] - s * n[k] + rwrows[i, pl.ds(k * _L, _L)]
                        a = jnp.abs(v)
                        acc = a if acc is None else acc + a
                    tot = _lanesum(acc)
                    vec = tot if vec is None else jnp.where(lane == j, tot, vec)
                outbuf[pl.ds(g * _L, _L)] = vec
                return 0

            lax.fori_loop(0, chunk // _L, group, 0)
            pltpu.sync_copy(outbuf, out_hbm.at[pl.ds(off, chunk)])

    return sc_kernel


def kernel(h, r, t, E_w, R_w, W_w):
    batch = h.shape[0]
    dim = E_w.shape[1]
    RW = _fuse_relations(R_w, W_w)
    sc = _make_sc_kernel(batch, dim, chunk=128)
    return sc(h, t, r, E_w, RW)
